# Initial kernel scaffold; baseline (speedup 1.0000x reference)
import jax
import jax.numpy as jnp
from jax.experimental import pallas as pl


def kernel(locs, imgs):
    raise NotImplementedError("write your pallas kernel here")


# speedup vs baseline: 3.1143x; 3.1143x over previous
import functools

import jax
import jax.numpy as jnp
from jax import lax
from jax.experimental import pallas as pl
from jax.experimental.pallas import tpu as pltpu
from jax.experimental.pallas import tpu_sc as plsc

B = 16
NF = 2
NT = 8
WIN = 15
IMG = 512
LANES = 16
NCORES = 2
NSUB = 16
NWORKERS = NCORES * NSUB
WPW = (B * NT) // NWORKERS
INV_SCALE = 512.0 / 15.0
HALF = WIN * 0.5
PR = 24
PC = 32


def _floorf(v):
    i = v.astype(jnp.int32)
    f = i.astype(jnp.float32)
    return jnp.where(f > v, f - 1.0, f)


def _sc_body(locs_hbm, grid_hbm, imgs_hbm, out_hbm,
             locs_v, grid_v, patches, outbuf, riv, wyv, sem_in, sem_out):
    cid = lax.axis_index("c")
    sid = lax.axis_index("s")
    wid = sid * NCORES + cid

    pltpu.sync_copy(locs_hbm, locs_v)
    pltpu.sync_copy(grid_hbm, grid_v)
    g = grid_v[:]
    zero16 = jnp.zeros((LANES,), jnp.int32)
    one16 = jnp.ones((LANES,), jnp.int32)

    metas = []
    in_copies = []
    for k in range(WPW):
        w = wid * WPW + k
        b = w // NT
        t = w % NT
        bv = jnp.full((LANES,), b, jnp.int32)
        tv = jnp.full((LANES,), t, jnp.int32)
        lxv = plsc.load_gather(locs_v, [bv, tv, zero16])
        lyv = plsc.load_gather(locs_v, [bv, tv, one16])
        cxmv = lxv * INV_SCALE
        cymv = lyv * INV_SCALE
        cbv = jnp.clip(_floorf((cxmv - 1.0) * HALF), 0.0,
                       float(IMG - 17)).astype(jnp.int32)
        rbv = jnp.clip(_floorf((cymv - 1.0) * HALF), 0.0,
                       float(IMG - 17)).astype(jnp.int32)
        c16v = (cbv // 16) * 16
        r8v = (rbv // 8) * 8
        c16 = pl.multiple_of(c16v[0], 16)
        r8 = pl.multiple_of(r8v[0], 8)
        metas.append((b, t, c16v, r8v, cxmv, cymv))
        for f in range(NF):
            in_copies.append(pltpu.async_copy(
                imgs_hbm.at[b, f, pl.ds(r8, PR), pl.ds(c16, PC)],
                patches.at[k * NF + f], sem_in))
    for cp in in_copies:
        cp.wait()

    out_copies = []
    for k in range(WPW):
        b, t, c16v, r8v, cxmv, cymv = metas[k]
        xv = (g + cxmv) * HALF
        x0f = _floorf(xv)
        x0c = jnp.clip(x0f, 0.0, float(IMG - 1))
        x1c = jnp.clip(x0f + 1.0, 0.0, float(IMG - 1))
        wx0 = x1c - xv
        wx1 = xv - x0c
        c0 = jnp.clip(x0c.astype(jnp.int32) - c16v, 0, PC - 1)
        c1 = jnp.clip(x1c.astype(jnp.int32) - c16v, 0, PC - 1)
        yv = (g + cymv) * HALF
        y0f = _floorf(yv)
        y0c = jnp.clip(y0f, 0.0, float(IMG - 1))
        y1c = jnp.clip(y0f + 1.0, 0.0, float(IMG - 1))
        riv[0, :] = jnp.clip(y0c.astype(jnp.int32) - r8v, 0, PR - 1)
        riv[1, :] = jnp.clip(y1c.astype(jnp.int32) - r8v, 0, PR - 1)
        wyv[0, :] = y1c - yv
        wyv[1, :] = yv - y0c
        for f in range(NF):
            kf = k * NF + f
            kfv = jnp.full((LANES,), kf, jnp.int32)

            def row_body(i, carry, _kfv=kfv, _c0=c0, _c1=c1,
                         _wx0=wx0, _wx1=wx1, _kf=kf):
                iv = jnp.full((LANES,), i, jnp.int32)
                r0v = plsc.load_gather(riv, [zero16, iv])
                r1v = plsc.load_gather(riv, [one16, iv])
                w0v = plsc.load_gather(wyv, [zero16, iv])
                w1v = plsc.load_gather(wyv, [one16, iv])
                pa = plsc.load_gather(patches, [_kfv, r0v, _c0])
                pb = plsc.load_gather(patches, [_kfv, r0v, _c1])
                pc_ = plsc.load_gather(patches, [_kfv, r1v, _c0])
                pd = plsc.load_gather(patches, [_kfv, r1v, _c1])
                row = (w0v * (_wx0 * pa + _wx1 * pb)
                       + w1v * (_wx0 * pc_ + _wx1 * pd))
                outbuf[_kf, i, :] = row
                return carry

            lax.fori_loop(0, WIN, row_body, 0)
            out_copies.append(pltpu.async_copy(
                outbuf.at[kf], out_hbm.at[b, f, t], sem_out))
    for cp in out_copies:
        cp.wait()


_sc_sample = functools.partial(
    pl.kernel,
    out_type=jax.ShapeDtypeStruct((B, NF, NT, WIN, LANES), jnp.float32),
    mesh=plsc.VectorSubcoreMesh(core_axis_name="c", subcore_axis_name="s"),
    compiler_params=pltpu.CompilerParams(
        use_tc_tiling_on_sc=False, needs_layout_passes=False),
    scratch_types=[
        pltpu.VMEM((B, NT, 2), jnp.float32),
        pltpu.VMEM((LANES,), jnp.float32),
        pltpu.VMEM((WPW * NF, PR, PC), jnp.float32),
        pltpu.VMEM((WPW * NF, WIN, LANES), jnp.float32),
        pltpu.VMEM((2, LANES), jnp.int32),
        pltpu.VMEM((2, LANES), jnp.float32),
        pltpu.SemaphoreType.DMA,
        pltpu.SemaphoreType.DMA,
    ],
)(_sc_body)


def kernel(locs, imgs):
    imgs4 = imgs.reshape(B, NF, IMG, IMG)
    xs = jnp.linspace(-1.0, 1.0, WIN, dtype=jnp.float32)
    g = jnp.concatenate([xs, jnp.zeros((1,), jnp.float32)])
    out = _sc_sample(locs, g, imgs4)
    return out[..., :WIN, None]
